# baseline (speedup 1.0000x reference)
import functools
import math

import jax
import jax.numpy as jnp
from jax import lax
from jax.experimental import pallas as pl
from jax.experimental.pallas import tpu as pltpu
from jax.experimental.pallas import tpu_sc as plsc

_PAD = 0
_SMOOTHING = 0.1
_CONFIDENCE = 1.0 - _SMOOTHING

_NC = 2
_NS = 16
_NW = _NC * _NS
_L = 16


def _make_sc_gather(n, v):
    bpw = n // _NW
    nch = bpw // _L
    mesh = plsc.VectorSubcoreMesh(core_axis_name="c", subcore_axis_name="s")

    @functools.partial(
        pl.kernel,
        mesh=mesh,
        out_type=jax.ShapeDtypeStruct((_NW, _L), jnp.float32),
        scratch_types=[
            pltpu.VMEM((bpw,), jnp.int32),
            pltpu.VMEM((bpw,), jnp.int32),
            pltpu.VMEM((bpw,), jnp.float32),
            pltpu.VMEM((_L,), jnp.float32),
            pltpu.SemaphoreType.DMA,
        ],
    )
    def sc_gather(xflat_hbm, tgt_hbm, out_hbm, tgtv, rowv, vals, stage, sem):
        wid = lax.axis_index("s") * _NC + lax.axis_index("c")
        base = wid * bpw
        pltpu.sync_copy(tgt_hbm.at[pl.ds(base, bpw)], tgtv)
        lanes = lax.broadcasted_iota(jnp.int32, (_L,), 0)
        for c in range(nch):
            t = tgtv[pl.ds(c * _L, _L)]
            ivec = (base + c * _L) + lanes
            rowv[pl.ds(c * _L, _L)] = ivec * v + t
        pltpu.async_copy(xflat_hbm.at[rowv], vals, sem).wait()
        acc = jnp.zeros((_L,), jnp.float32)
        for c in range(nch):
            t = tgtv[pl.ds(c * _L, _L)]
            x_t = vals[pl.ds(c * _L, _L)]
            acc = acc + jnp.where(t != _PAD, x_t, 0.0)
        stage[...] = acc
        pltpu.sync_copy(stage, out_hbm.at[wid])

    return sc_gather


def _tc_body(tgt_ref, x_ref, scg_ref, out_ref, acc_ref, nv_ref, x0_ref,
             *, ncb, smooth):
    j = pl.program_id(0)
    t = tgt_ref[...]
    validf = (t != _PAD).astype(jnp.float32)
    xb = x_ref[...]

    contrib = jnp.sum(xb * validf)

    @pl.when(j == 0)
    def _init():
        nv_ref[0] = jnp.sum(validf)
        x0_ref[0] = jnp.sum(validf * xb[:, 0:1])
        acc_ref[0] = contrib

    @pl.when(j > 0)
    def _accum():
        acc_ref[0] = acc_ref[0] + contrib

    @pl.when(j == ncb - 1)
    def _finish():
        v = x_ref.shape[1] * ncb
        k_const = ((v - 2) * smooth * math.log(smooth)
                   + _CONFIDENCE * math.log(_CONFIDENCE))
        g = jnp.sum(scg_ref[...])
        nv = nv_ref[0]
        out_ref[0, 0] = (-smooth * acc_ref[0] + smooth * x0_ref[0]
                         + (smooth - _CONFIDENCE) * g + k_const * nv) / nv


def kernel(x, target):
    x2 = x.reshape(-1, x.shape[-1])
    n, v = x2.shape
    tgt = target.reshape(-1).astype(jnp.int32)
    smooth = _SMOOTHING / (v - 2)

    xflat = x2.reshape(n * v)
    sc_partials = _make_sc_gather(n, v)(xflat, tgt)

    cb = 1024
    ncb = v // cb
    out = pl.pallas_call(
        functools.partial(_tc_body, ncb=ncb, smooth=smooth),
        grid=(ncb,),
        in_specs=[
            pl.BlockSpec((n, 1), lambda j: (0, 0)),
            pl.BlockSpec((n, cb), lambda j: (0, j)),
            pl.BlockSpec((_NW, _L), lambda j: (0, 0)),
        ],
        out_specs=pl.BlockSpec(memory_space=pltpu.SMEM),
        out_shape=jax.ShapeDtypeStruct((1, 1), jnp.float32),
        scratch_shapes=[
            pltpu.SMEM((1,), jnp.float32),
            pltpu.SMEM((1,), jnp.float32),
            pltpu.SMEM((1,), jnp.float32),
        ],
    )(tgt.reshape(n, 1), x2, sc_partials)
    return out[0, 0]


# speedup vs baseline: 2.7332x; 2.7332x over previous
import functools
import math

import jax
import jax.numpy as jnp
from jax import lax
from jax.experimental import pallas as pl
from jax.experimental.pallas import tpu as pltpu
from jax.experimental.pallas import tpu_sc as plsc

_PAD = 0
_SMOOTHING = 0.1
_CONFIDENCE = 1.0 - _SMOOTHING

_NC = 2
_NS = 16
_NW = _NC * _NS
_L = 16


def _make_sc_gather(n, v):
    bpw = n // _NW
    batch = 64
    mesh = plsc.VectorSubcoreMesh(core_axis_name="c", subcore_axis_name="s")

    @functools.partial(
        pl.kernel,
        mesh=mesh,
        out_type=jax.ShapeDtypeStruct((_NW, _L), jnp.float32),
        scratch_types=[
            pltpu.VMEM((bpw,), jnp.int32),
            pltpu.VMEM((batch, 8, 128), jnp.float32),
            pltpu.VMEM((_L,), jnp.float32),
            pltpu.SemaphoreType.DMA,
        ],
    )
    def sc_gather(x_hbm, tgt_hbm, out_hbm, tgtv, bufs, stage, sem):
        wid = lax.axis_index("s") * _NC + lax.axis_index("c")
        base = wid * bpw
        pltpu.sync_copy(tgt_hbm.at[pl.ds(base, bpw)], tgtv)
        lanes = lax.broadcasted_iota(jnp.int32, (_L,), 0)
        acc = jnp.zeros((_L,), jnp.float32)
        for b0 in range(0, bpw, batch):
            copies = []
            for c in range(batch // _L):
                tv = tgtv[pl.ds(b0 + c * _L, _L)]
                for q in range(_L):
                    k = b0 + c * _L + q
                    t = tv[q]
                    col0 = pl.multiple_of(jnp.bitwise_and(t, -128), 128)
                    cp = pltpu.make_async_copy(
                        x_hbm.at[pl.ds(base + (k & ~7), 8), pl.ds(col0, 128)],
                        bufs.at[k - b0], sem)
                    cp.start()
                    copies.append(cp)
            for cp in copies:
                cp.wait()
            for c in range(batch // _L):
                tv = tgtv[pl.ds(b0 + c * _L, _L)]
                for q in range(_L):
                    k = b0 + c * _L + q
                    t = tv[q]
                    w0 = jnp.bitwise_and(t, 127 - (_L - 1))
                    vals = bufs[k - b0, k & 7, pl.ds(w0, _L)]
                    sel = lanes == jnp.bitwise_and(t, _L - 1)
                    validf = (t != _PAD).astype(jnp.float32)
                    acc = acc + jnp.where(sel, vals, 0.0) * validf
        stage[...] = acc
        pltpu.sync_copy(stage, out_hbm.at[wid])

    return sc_gather


def _tc_body(tgt_ref, x_ref, scg_ref, out_ref, acc_ref, nv_ref, x0_ref,
             *, ncb, smooth):
    j = pl.program_id(0)
    t = tgt_ref[...]
    validf = (t != _PAD).astype(jnp.float32)
    xb = x_ref[...]

    contrib = jnp.sum(xb * validf)

    @pl.when(j == 0)
    def _init():
        nv_ref[0] = jnp.sum(validf)
        x0_ref[0] = jnp.sum(validf * xb[:, 0:1])
        acc_ref[0] = contrib

    @pl.when(j > 0)
    def _accum():
        acc_ref[0] = acc_ref[0] + contrib

    @pl.when(j == ncb - 1)
    def _finish():
        v = x_ref.shape[1] * ncb
        k_const = ((v - 2) * smooth * math.log(smooth)
                   + _CONFIDENCE * math.log(_CONFIDENCE))
        g = jnp.sum(scg_ref[...])
        nv = nv_ref[0]
        out_ref[0, 0] = (-smooth * acc_ref[0] + smooth * x0_ref[0]
                         + (smooth - _CONFIDENCE) * g + k_const * nv) / nv


def kernel(x, target):
    x2 = x.reshape(-1, x.shape[-1])
    n, v = x2.shape
    tgt = target.reshape(-1).astype(jnp.int32)
    smooth = _SMOOTHING / (v - 2)

    sc_partials = _make_sc_gather(n, v)(x2, tgt)

    cb = 1024
    ncb = v // cb
    out = pl.pallas_call(
        functools.partial(_tc_body, ncb=ncb, smooth=smooth),
        grid=(ncb,),
        in_specs=[
            pl.BlockSpec((n, 1), lambda j: (0, 0)),
            pl.BlockSpec((n, cb), lambda j: (0, j)),
            pl.BlockSpec((_NW, _L), lambda j: (0, 0)),
        ],
        out_specs=pl.BlockSpec(memory_space=pltpu.SMEM),
        out_shape=jax.ShapeDtypeStruct((1, 1), jnp.float32),
        scratch_shapes=[
            pltpu.SMEM((1,), jnp.float32),
            pltpu.SMEM((1,), jnp.float32),
            pltpu.SMEM((1,), jnp.float32),
        ],
    )(tgt.reshape(n, 1), x2, sc_partials)
    return out[0, 0]


# speedup vs baseline: 3.0259x; 1.1071x over previous
import functools
import math

import jax
import jax.numpy as jnp
from jax import lax
from jax.experimental import pallas as pl
from jax.experimental.pallas import tpu as pltpu
from jax.experimental.pallas import tpu_sc as plsc

_PAD = 0
_SMOOTHING = 0.1
_CONFIDENCE = 1.0 - _SMOOTHING

_NC = 2
_NS = 16
_NW = _NC * _NS
_L = 16


def _make_sc_gather(n, v):
    bpw = n // _NW
    batch = 64
    mesh = plsc.VectorSubcoreMesh(core_axis_name="c", subcore_axis_name="s")

    @functools.partial(
        pl.kernel,
        mesh=mesh,
        out_type=jax.ShapeDtypeStruct((_NW, _L), jnp.float32),
        scratch_types=[
            pltpu.VMEM((bpw,), jnp.int32),
            pltpu.VMEM((batch, 8, 128), jnp.float32),
            pltpu.VMEM((_L,), jnp.float32),
            pltpu.SemaphoreType.DMA,
        ],
    )
    def sc_gather(x_hbm, tgt_hbm, out_hbm, tgtv, bufs, stage, sem):
        wid = lax.axis_index("s") * _NC + lax.axis_index("c")
        base = wid * bpw
        pltpu.sync_copy(tgt_hbm.at[pl.ds(base, bpw)], tgtv)
        lanes = lax.broadcasted_iota(jnp.int32, (_L,), 0)
        acc = jnp.zeros((_L,), jnp.float32)
        for b0 in range(0, bpw, batch):
            copies = []
            for c in range(batch // _L):
                tv = tgtv[pl.ds(b0 + c * _L, _L)]
                for q in range(_L):
                    k = b0 + c * _L + q
                    t = tv[q]
                    col0 = pl.multiple_of(jnp.bitwise_and(t, -128), 128)
                    cp = pltpu.make_async_copy(
                        x_hbm.at[pl.ds(base + (k & ~7), 8), pl.ds(col0, 128)],
                        bufs.at[k - b0], sem)
                    cp.start()
                    copies.append(cp)
            for cp in copies:
                cp.wait()
            for c in range(batch // _L):
                tv = tgtv[pl.ds(b0 + c * _L, _L)]
                for q in range(_L):
                    k = b0 + c * _L + q
                    t = tv[q]
                    w0 = jnp.bitwise_and(t, 127 - (_L - 1))
                    vals = bufs[k - b0, k & 7, pl.ds(w0, _L)]
                    sel = lanes == jnp.bitwise_and(t, _L - 1)
                    validf = (t != _PAD).astype(jnp.float32)
                    acc = acc + jnp.where(sel, vals, 0.0) * validf
        stage[...] = acc
        pltpu.sync_copy(stage, out_hbm.at[wid])

    return sc_gather


def _tc_body(tgt_ref, x_ref, out_ref, nv_ref, acc_ref, *, ncb, smooth):
    j = pl.program_id(0)
    t = tgt_ref[...]
    validf = (t != _PAD).astype(jnp.float32)
    xb = x_ref[...]

    rs = jnp.sum(xb, axis=1, keepdims=True)
    contrib = jnp.sum(validf * rs)

    @pl.when(j == 0)
    def _init():
        nv_ref[0, 0] = jnp.sum(validf)
        acc_ref[0] = contrib - jnp.sum(validf * xb[:, 0:1])

    @pl.when(j > 0)
    def _accum():
        acc_ref[0] = acc_ref[0] + contrib

    @pl.when(j == ncb - 1)
    def _finish():
        v = x_ref.shape[1] * ncb
        k_const = ((v - 2) * smooth * math.log(smooth)
                   + _CONFIDENCE * math.log(_CONFIDENCE))
        nv = nv_ref[0, 0]
        out_ref[0, 0] = -smooth * acc_ref[0] + k_const * nv


def kernel(x, target):
    x2 = x.reshape(-1, x.shape[-1])
    n, v = x2.shape
    tgt = target.reshape(-1).astype(jnp.int32)
    smooth = _SMOOTHING / (v - 2)

    sc_partials = _make_sc_gather(n, v)(x2, tgt)

    cb = 1024
    ncb = v // cb
    acc, nv = pl.pallas_call(
        functools.partial(_tc_body, ncb=ncb, smooth=smooth),
        grid=(ncb,),
        in_specs=[
            pl.BlockSpec((n, 1), lambda j: (0, 0)),
            pl.BlockSpec((n, cb), lambda j: (0, j)),
        ],
        out_specs=[
            pl.BlockSpec(memory_space=pltpu.SMEM),
            pl.BlockSpec(memory_space=pltpu.SMEM),
        ],
        out_shape=[
            jax.ShapeDtypeStruct((1, 1), jnp.float32),
            jax.ShapeDtypeStruct((1, 1), jnp.float32),
        ],
        scratch_shapes=[
            pltpu.SMEM((1,), jnp.float32),
        ],
    )(tgt.reshape(n, 1), x2)
    g = jnp.sum(sc_partials)
    nv = nv[0, 0]
    return (acc[0, 0] + (smooth - _CONFIDENCE) * g) / nv


# speedup vs baseline: 3.0746x; 1.0161x over previous
import functools
import math

import jax
import jax.numpy as jnp
from jax import lax
from jax.experimental import pallas as pl
from jax.experimental.pallas import tpu as pltpu
from jax.experimental.pallas import tpu_sc as plsc

_PAD = 0
_SMOOTHING = 0.1
_CONFIDENCE = 1.0 - _SMOOTHING

_NC = 2
_NS = 16
_NW = _NC * _NS
_L = 16


def _make_sc_gather(n, v):
    bpw = n // _NW
    batch = 64
    mesh = plsc.VectorSubcoreMesh(core_axis_name="c", subcore_axis_name="s")

    @functools.partial(
        pl.kernel,
        mesh=mesh,
        out_type=jax.ShapeDtypeStruct((_NW, _L), jnp.float32),
        scratch_types=[
            pltpu.VMEM((bpw,), jnp.int32),
            pltpu.VMEM((batch, 8, 128), jnp.float32),
            pltpu.VMEM((_L,), jnp.float32),
            pltpu.SemaphoreType.DMA,
        ],
    )
    def sc_gather(x_hbm, tgt_hbm, out_hbm, tgtv, bufs, stage, sem):
        wid = lax.axis_index("s") * _NC + lax.axis_index("c")
        base = wid * bpw
        pltpu.sync_copy(tgt_hbm.at[pl.ds(base, bpw)], tgtv)
        lanes = lax.broadcasted_iota(jnp.int32, (_L,), 0)
        acc = jnp.zeros((_L,), jnp.float32)
        for b0 in range(0, bpw, batch):
            copies = []
            for c in range(batch // _L):
                tv = tgtv[pl.ds(b0 + c * _L, _L)]
                for q in range(_L):
                    k = b0 + c * _L + q
                    t = tv[q]
                    col0 = pl.multiple_of(jnp.bitwise_and(t, -128), 128)
                    cp = pltpu.make_async_copy(
                        x_hbm.at[pl.ds(base + (k & ~7), 8), pl.ds(col0, 128)],
                        bufs.at[k - b0], sem)
                    cp.start()
                    copies.append(cp)
            for cp in copies:
                cp.wait()
            for c in range(batch // _L):
                tv = tgtv[pl.ds(b0 + c * _L, _L)]
                for q in range(_L):
                    k = b0 + c * _L + q
                    t = tv[q]
                    w0 = jnp.bitwise_and(t, 127 - (_L - 1))
                    vals = bufs[k - b0, k & 7, pl.ds(w0, _L)]
                    sel = lanes == jnp.bitwise_and(t, _L - 1)
                    validf = (t != _PAD).astype(jnp.float32)
                    acc = acc + jnp.where(sel, vals, 0.0) * validf
        stage[...] = acc
        pltpu.sync_copy(stage, out_hbm.at[wid])

    return sc_gather


def _tc_body(tgt_ref, x_ref, out_ref, nv_ref, acc_ref, nvacc_ref,
             *, nrb, smooth):
    j = pl.program_id(0)
    t = tgt_ref[...]
    validf = (t != _PAD).astype(jnp.float32)
    xb = x_ref[...]

    rs = jnp.sum(xb, axis=1, keepdims=True)
    contrib = jnp.sum(validf * (rs - xb[:, 0:1]))
    nv_part = jnp.sum(validf)

    @pl.when(j == 0)
    def _init():
        acc_ref[0] = contrib
        nvacc_ref[0] = nv_part

    @pl.when(j > 0)
    def _accum():
        acc_ref[0] = acc_ref[0] + contrib
        nvacc_ref[0] = nvacc_ref[0] + nv_part

    @pl.when(j == nrb - 1)
    def _finish():
        v = x_ref.shape[1]
        k_const = ((v - 2) * smooth * math.log(smooth)
                   + _CONFIDENCE * math.log(_CONFIDENCE))
        nv = nvacc_ref[0]
        nv_ref[0, 0] = nv
        out_ref[0, 0] = -smooth * acc_ref[0] + k_const * nv


def kernel(x, target):
    x2 = x.reshape(-1, x.shape[-1])
    n, v = x2.shape
    tgt = target.reshape(-1).astype(jnp.int32)
    smooth = _SMOOTHING / (v - 2)

    sc_partials = _make_sc_gather(n, v)(x2, tgt)

    rb = 128
    nrb = n // rb
    acc, nv = pl.pallas_call(
        functools.partial(_tc_body, nrb=nrb, smooth=smooth),
        grid=(nrb,),
        in_specs=[
            pl.BlockSpec((rb, 1), lambda j: (j, 0)),
            pl.BlockSpec((rb, v), lambda j: (j, 0)),
        ],
        out_specs=[
            pl.BlockSpec(memory_space=pltpu.SMEM),
            pl.BlockSpec(memory_space=pltpu.SMEM),
        ],
        out_shape=[
            jax.ShapeDtypeStruct((1, 1), jnp.float32),
            jax.ShapeDtypeStruct((1, 1), jnp.float32),
        ],
        scratch_shapes=[
            pltpu.SMEM((1,), jnp.float32),
            pltpu.SMEM((1,), jnp.float32),
        ],
    )(tgt.reshape(n, 1), x2)
    g = jnp.sum(sc_partials)
    nv = nv[0, 0]
    return (acc[0, 0] + (smooth - _CONFIDENCE) * g) / nv
